# baseline (speedup 1.0000x reference)
import functools

import jax
import jax.numpy as jnp
from jax import lax
from jax.experimental import pallas as pl
from jax.experimental.pallas import tpu as pltpu
from jax.experimental.pallas import tpu_sc as plsc

B, L, V, D = 4096, 50, 100000, 300
LANES = 16
NW = 32
ITEMS = B // NW
W = 64
NG = V * D // W
CH = 6
CW = CH * W
NCLS = W // 4
ACC = CW + 4
ACCW = NCLS * CW
NWIN = (D + LANES - 1) // LANES
OFFS = tuple(min(LANES * j, D - LANES) for j in range(NWIN))
INV_L = 1.0 / L
TOFFS = (0, 16, 32, 34)


def _pool_body(x_hbm, tab_hbm, out_hbm, xv, gidx, cbuf, chunks, accbuf, hv,
               sem0, sem1):
    c = lax.axis_index("c")
    s = lax.axis_index("s")
    wid = s * 2 + c
    pltpu.sync_copy(x_hbm.at[pl.ds(wid * (ITEMS * L), ITEMS * L)], xv)
    iota = lax.broadcasted_iota(jnp.int32, (LANES,), 0)
    sems = (sem0, sem1)

    def build(i, k):
        for off in TOFFS:
            t = xv[pl.ds(L * i + off, LANES)]
            m44 = t * 44
            g0 = t * 4 + (m44 >> 6)
            cbuf[k, pl.ds(off, LANES)] = (
                ((m44 & (W - 1)) >> 2) * CW).astype(jnp.float32)
            for j in range(CH):
                gidx[k, j, pl.ds(off, LANES)] = jnp.minimum(g0 + j, NG - 1)

    def start(i, k):
        for q in range(CH):
            pltpu.async_copy(
                tab_hbm.at[gidx.at[k, q]], chunks.at[k, q], sems[k])

    def wait(k):
        for q in range(CH):
            pltpu.make_async_copy(
                tab_hbm.at[gidx.at[k, q]], chunks.at[k, q], sems[k]).wait()

    def accumulate(i, k):
        def tbody(r, _):
            bspl = plsc.load_gather(cbuf.at[k], [r + iota * 0])
            bidx = bspl.astype(jnp.int32) + iota
            for j in range(CH):
                for m in range(W // LANES):
                    v = chunks[k, j, r, pl.ds(LANES * m, LANES)]
                    plsc.addupdate_scatter(
                        accbuf, [bidx + (W * j + LANES * m)], v)
            return 0

        lax.fori_loop(0, L, tbody, 0, unroll=False)

    zvec = jnp.zeros((LANES,), jnp.float32)

    def zero_acc():
        def zbody(w, _):
            accbuf[pl.ds(LANES * w, LANES)] = zvec
            return 0

        lax.fori_loop(0, ACCW // LANES, zbody, 0, unroll=8)

    def finalize(i):
        for j in range(NWIN):
            acc = accbuf[pl.ds(OFFS[j], LANES)]
            for cc in range(1, NCLS):
                acc = acc + accbuf[pl.ds(ACC * cc + OFFS[j], LANES)]
            hv[pl.ds(D * i + OFFS[j], LANES)] = acc * INV_L
        zero_acc()

    zero_acc()
    build(0, 0)
    start(0, 0)

    def gbody(g, _):
        for k in range(2):
            i = 2 * g + k
            wait(k)

            @pl.when(i + 1 < ITEMS)
            def _():
                build(i + 1, 1 - k)
                start(i + 1, 1 - k)

            accumulate(i, k)
            finalize(i)
        return 0

    lax.fori_loop(0, ITEMS // 2, gbody, 0, unroll=False)
    pltpu.sync_copy(hv, out_hbm.at[pl.ds(wid * (ITEMS * D), ITEMS * D)])


def _make_pool():
    mesh = plsc.VectorSubcoreMesh(core_axis_name="c", subcore_axis_name="s")
    return functools.partial(
        pl.kernel,
        mesh=mesh,
        compiler_params=pltpu.CompilerParams(
            use_tc_tiling_on_sc=False, needs_layout_passes=False),
        out_type=jax.ShapeDtypeStruct((B * D,), jnp.float32),
        scratch_types=[
            pltpu.VMEM((ITEMS * L,), jnp.int32),
            pltpu.VMEM((2, CH, L), jnp.int32),
            pltpu.VMEM((2, 64), jnp.float32),
            pltpu.VMEM((2, CH, L, W), jnp.float32),
            pltpu.VMEM((ACCW,), jnp.float32),
            pltpu.VMEM((ITEMS * D,), jnp.float32),
            pltpu.SemaphoreType.DMA,
            pltpu.SemaphoreType.DMA,
        ],
    )(_pool_body)


_pool = _make_pool()


def _pooled(x, table):
    x_flat = x.astype(jnp.int32).reshape(B * L)
    tabw = table.reshape(NG, W)
    return _pool(x_flat, tabw).reshape(B, D)


def _mlp_body(h_ref, w1_ref, b1_ref, w2_ref, b2_ref, w3_ref, b3_ref, o_ref):
    h = h_ref[...]
    z = jnp.maximum(jnp.dot(h, w1_ref[...],
                            preferred_element_type=jnp.float32) + b1_ref[...], 0.0)
    z = jnp.maximum(jnp.dot(z, w2_ref[...],
                            preferred_element_type=jnp.float32) + b2_ref[...], 0.0)
    o_ref[...] = jnp.dot(z, w3_ref[...],
                         preferred_element_type=jnp.float32) + b3_ref[...]


def _mlp(h, W1, b1, W2, b2, W3, b3):
    b1, b2, b3 = b1[None, :], b2[None, :], b3[None, :]
    BLK = 512
    grid = (B // BLK,)
    full = lambda shape: pl.BlockSpec(shape, lambda i: (0, 0))
    return pl.pallas_call(
        _mlp_body,
        grid=grid,
        in_specs=[
            pl.BlockSpec((BLK, D), lambda i: (i, 0)),
            full(W1.shape), full(b1.shape), full(W2.shape), full(b2.shape),
            full(W3.shape), full(b3.shape),
        ],
        out_specs=pl.BlockSpec((BLK, 1), lambda i: (i, 0)),
        out_shape=jax.ShapeDtypeStruct((B, 1), jnp.float32),
    )(h, W1, b1, W2, b2, W3, b3)


def kernel(x, table, W1, b1, W2, b2, W3, b3):
    h = _pooled(x, table)
    return _mlp(h, W1, b1, W2, b2, W3, b3)


# speedup vs baseline: 1.1172x; 1.1172x over previous
import functools

import jax
import jax.numpy as jnp
from jax import lax
from jax.experimental import pallas as pl
from jax.experimental.pallas import tpu as pltpu
from jax.experimental.pallas import tpu_sc as plsc

B, L, V, D = 4096, 50, 100000, 300
LANES = 16
NW = 32
ITEMS = B // NW
NG = V * D // LANES
CH = 20
GI = 1024
NSTR = 8
CW = CH * LANES
ACC = 324
NBUF = 4
NWIN = (D + LANES - 1) // LANES
OFFS = tuple(min(LANES * j, D - LANES) for j in range(NWIN))
INV_L = 1.0 / L
TOFFS = (0, 16, 32, 34)


def _pool_body(x_hbm, tab_hbm, out_hbm, xv, gidx, cbuf, chunks, accbuf, hv,
               *sems):
    c = lax.axis_index("c")
    s = lax.axis_index("s")
    wid = s * 2 + c
    pltpu.sync_copy(x_hbm.at[pl.ds(wid * (ITEMS * L), ITEMS * L)], xv)
    iota = lax.broadcasted_iota(jnp.int32, (LANES,), 0)

    def build(i, k):
        for off in TOFFS:
            t = xv[pl.ds(L * i + off, LANES)]
            m12 = t * 12
            g0 = t * 18 + (m12 >> 4)
            cbuf[k, pl.ds(off, LANES)] = (
                ((m12 & 15) >> 2) * CW).astype(jnp.float32)
            for j in range(CH):
                gidx[k, pl.ds(L * j + off, LANES)] = jnp.minimum(
                    g0 + j, NG - 1)
        pad = wid * LANES + iota
        gidx[k, pl.ds(CH * L, LANES)] = pad
        gidx[k, pl.ds(CH * L + 8, LANES)] = pad

    def start(i, k):
        for q in range(NSTR):
            pltpu.async_copy(
                tab_hbm.at[gidx.at[k, pl.ds(128 * q, 128)]],
                chunks.at[k, pl.ds(128 * q, 128)], sems[k])

    def wait(k):
        for q in range(NSTR):
            pltpu.make_async_copy(
                tab_hbm.at[gidx.at[k, pl.ds(128 * q, 128)]],
                chunks.at[k, pl.ds(128 * q, 128)], sems[k]).wait()

    def accumulate(i, k):
        def tbody(r, _):
            bspl = plsc.load_gather(cbuf.at[k], [r + iota * 0])
            bidx = bspl.astype(jnp.int32) + iota
            for j in range(CH):
                v = chunks[k, L * j + r, :]
                plsc.addupdate_scatter(accbuf, [bidx + LANES * j], v)
            return 0

        lax.fori_loop(0, L, tbody, 0, unroll=False)

    def finalize(i):
        for j in range(NWIN):
            acc = accbuf[pl.ds(OFFS[j], LANES)]
            for cc in range(1, 4):
                acc = acc + accbuf[pl.ds(ACC * cc + OFFS[j], LANES)]
            hv[pl.ds(D * i + OFFS[j], LANES)] = acc * INV_L
        z = jnp.zeros((LANES,), jnp.float32)
        for w in range(4 * CW // LANES):
            accbuf[pl.ds(LANES * w, LANES)] = z

    z = jnp.zeros((LANES,), jnp.float32)
    for w in range(4 * CW // LANES):
        accbuf[pl.ds(LANES * w, LANES)] = z

    for k in range(NBUF - 1):
        build(k, k)
        start(k, k)

    def gbody(g, _):
        for k in range(NBUF):
            i = NBUF * g + k
            wait(k)

            @pl.when(i + (NBUF - 1) < ITEMS)
            def _():
                build(i + (NBUF - 1), (k + NBUF - 1) % NBUF)
                start(i + (NBUF - 1), (k + NBUF - 1) % NBUF)

            accumulate(i, k)
            finalize(i)
        return 0

    lax.fori_loop(0, ITEMS // NBUF, gbody, 0, unroll=False)
    pltpu.sync_copy(hv, out_hbm.at[pl.ds(wid * (ITEMS * D), ITEMS * D)])


def _make_pool():
    mesh = plsc.VectorSubcoreMesh(core_axis_name="c", subcore_axis_name="s")
    return functools.partial(
        pl.kernel,
        mesh=mesh,
        compiler_params=pltpu.CompilerParams(
            use_tc_tiling_on_sc=False, needs_layout_passes=False),
        out_type=jax.ShapeDtypeStruct((B * D,), jnp.float32),
        scratch_types=[
            pltpu.VMEM((ITEMS * L,), jnp.int32),
            pltpu.VMEM((NBUF, GI), jnp.int32),
            pltpu.VMEM((NBUF, 64), jnp.float32),
            pltpu.VMEM((NBUF, GI, LANES), jnp.float32),
            pltpu.VMEM((4 * CW,), jnp.float32),
            pltpu.VMEM((ITEMS * D,), jnp.float32),
        ] + [pltpu.SemaphoreType.DMA] * NBUF,
    )(_pool_body)


_pool = _make_pool()


def _pooled(x, table):
    x_flat = x.astype(jnp.int32).reshape(B * L)
    tab16 = table.reshape(NG, LANES)
    return _pool(x_flat, tab16).reshape(B, D)


def _mlp_body(h_ref, w1_ref, b1_ref, w2_ref, b2_ref, w3_ref, b3_ref, o_ref):
    h = h_ref[...]
    z = jnp.maximum(jnp.dot(h, w1_ref[...],
                            preferred_element_type=jnp.float32) + b1_ref[...], 0.0)
    z = jnp.maximum(jnp.dot(z, w2_ref[...],
                            preferred_element_type=jnp.float32) + b2_ref[...], 0.0)
    o_ref[...] = jnp.dot(z, w3_ref[...],
                         preferred_element_type=jnp.float32) + b3_ref[...]


def _mlp(h, W1, b1, W2, b2, W3, b3):
    BLK = 512
    grid = (B // BLK,)
    full = lambda shape: pl.BlockSpec(shape, lambda i: (0, 0))
    return pl.pallas_call(
        _mlp_body,
        grid=grid,
        in_specs=[
            pl.BlockSpec((BLK, D), lambda i: (i, 0)),
            full(W1.shape), full(b1.shape), full(W2.shape), full(b2.shape),
            full(W3.shape), full(b3.shape),
        ],
        out_specs=pl.BlockSpec((BLK, 1), lambda i: (i, 0)),
        out_shape=jax.ShapeDtypeStruct((B, 1), jnp.float32),
    )(h, W1, b1, W2, b2, W3, b3)


def kernel(x, table, W1, b1, W2, b2, W3, b3):
    h = _pooled(x, table)
    return _mlp(h, W1, b1[None, :], W2, b2[None, :], W3, b3[None, :])


# speedup vs baseline: 1.1313x; 1.0126x over previous
import functools

import jax
import jax.numpy as jnp
from jax import lax
from jax.experimental import pallas as pl
from jax.experimental.pallas import tpu as pltpu
from jax.experimental.pallas import tpu_sc as plsc

B, L, V, D = 4096, 50, 100000, 300
LANES = 16
NW = 32
ITEMS = B // NW
NG = V * D // LANES
CH = 20
GI = 1000
NSTR = 5
CW = CH * LANES
ACC = 324
NWIN = (D + LANES - 1) // LANES
OFFS = tuple(min(LANES * j, D - LANES) for j in range(NWIN))
INV_L = 1.0 / L
TOFFS = (0, 16, 32, 34)


def _pool_body(x_hbm, tab_hbm, out_hbm, xv, gidx, cbuf, chunks, accbuf, hv,
               sem0, sem1):
    c = lax.axis_index("c")
    s = lax.axis_index("s")
    wid = s * 2 + c
    pltpu.sync_copy(x_hbm.at[pl.ds(wid * (ITEMS * L), ITEMS * L)], xv)
    iota = lax.broadcasted_iota(jnp.int32, (LANES,), 0)
    sems = (sem0, sem1)

    def build(i, k):
        for off in TOFFS:
            t = xv[pl.ds(L * i + off, LANES)]
            m12 = t * 12
            g0 = t * 18 + (m12 >> 4)
            cbuf[k, pl.ds(off, LANES)] = (
                ((m12 & 15) >> 2) * CW).astype(jnp.float32)
            for j in range(CH):
                gidx[k, pl.ds(L * j + off, LANES)] = jnp.minimum(
                    g0 + j, NG - 1)

    def start(i, k):
        for q in range(NSTR):
            pltpu.async_copy(
                tab_hbm.at[gidx.at[k, pl.ds(200 * q, 200)]],
                chunks.at[k, pl.ds(200 * q, 200)], sems[k])

    def wait(k):
        for q in range(NSTR):
            pltpu.make_async_copy(
                tab_hbm.at[gidx.at[k, pl.ds(200 * q, 200)]],
                chunks.at[k, pl.ds(200 * q, 200)], sems[k]).wait()

    def accumulate(i, k):
        def tbody(r, _):
            bspl = plsc.load_gather(cbuf.at[k], [r + iota * 0])
            bidx = bspl.astype(jnp.int32) + iota
            for j in range(CH):
                v = chunks[k, L * j + r, :]
                plsc.addupdate_scatter(accbuf, [bidx + LANES * j], v)
            return 0

        lax.fori_loop(0, L, tbody, 0, unroll=False)

    def finalize(i):
        for j in range(NWIN):
            acc = accbuf[pl.ds(OFFS[j], LANES)]
            for cc in range(1, 4):
                acc = acc + accbuf[pl.ds(ACC * cc + OFFS[j], LANES)]
            hv[pl.ds(D * i + OFFS[j], LANES)] = acc * INV_L
        z = jnp.zeros((LANES,), jnp.float32)
        for w in range(4 * CW // LANES):
            accbuf[pl.ds(LANES * w, LANES)] = z

    z = jnp.zeros((LANES,), jnp.float32)
    for w in range(4 * CW // LANES):
        accbuf[pl.ds(LANES * w, LANES)] = z

    build(0, 0)
    start(0, 0)

    def gbody(g, _):
        for k in range(2):
            i = 2 * g + k
            wait(k)

            @pl.when(i + 1 < ITEMS)
            def _():
                build(i + 1, 1 - k)
                start(i + 1, 1 - k)

            accumulate(i, k)
            finalize(i)
        return 0

    lax.fori_loop(0, ITEMS // 2, gbody, 0, unroll=False)
    pltpu.sync_copy(hv, out_hbm.at[pl.ds(wid * (ITEMS * D), ITEMS * D)])


def _make_pool():
    mesh = plsc.VectorSubcoreMesh(core_axis_name="c", subcore_axis_name="s")
    return functools.partial(
        pl.kernel,
        mesh=mesh,
        compiler_params=pltpu.CompilerParams(
            use_tc_tiling_on_sc=False, needs_layout_passes=False),
        out_type=jax.ShapeDtypeStruct((B * D,), jnp.float32),
        scratch_types=[
            pltpu.VMEM((ITEMS * L,), jnp.int32),
            pltpu.VMEM((2, GI), jnp.int32),
            pltpu.VMEM((2, 64), jnp.float32),
            pltpu.VMEM((2, GI, LANES), jnp.float32),
            pltpu.VMEM((4 * CW,), jnp.float32),
            pltpu.VMEM((ITEMS * D,), jnp.float32),
            pltpu.SemaphoreType.DMA,
            pltpu.SemaphoreType.DMA,
        ],
    )(_pool_body)


_pool = _make_pool()


def _pooled(x, table):
    x_flat = x.astype(jnp.int32).reshape(B * L)
    tab16 = table.reshape(NG, LANES)
    return _pool(x_flat, tab16).reshape(B, D)


def _mlp_body(h_ref, w1_ref, b1_ref, w2_ref, b2_ref, w3_ref, b3_ref, o_ref):
    h = h_ref[...]
    z = jnp.maximum(jnp.dot(h, w1_ref[...],
                            preferred_element_type=jnp.float32) + b1_ref[...], 0.0)
    z = jnp.maximum(jnp.dot(z, w2_ref[...],
                            preferred_element_type=jnp.float32) + b2_ref[...], 0.0)
    o_ref[...] = jnp.dot(z, w3_ref[...],
                         preferred_element_type=jnp.float32) + b3_ref[...]


def _mlp(h, W1, b1, W2, b2, W3, b3):
    BLK = 512
    grid = (B // BLK,)
    full = lambda shape: pl.BlockSpec(shape, lambda i: (0, 0))
    return pl.pallas_call(
        _mlp_body,
        grid=grid,
        in_specs=[
            pl.BlockSpec((BLK, D), lambda i: (i, 0)),
            full(W1.shape), full(b1.shape), full(W2.shape), full(b2.shape),
            full(W3.shape), full(b3.shape),
        ],
        out_specs=pl.BlockSpec((BLK, 1), lambda i: (i, 0)),
        out_shape=jax.ShapeDtypeStruct((B, 1), jnp.float32),
    )(h, W1, b1, W2, b2, W3, b3)


def kernel(x, table, W1, b1, W2, b2, W3, b3):
    h = _pooled(x, table)
    return _mlp(h, W1, b1[None, :], W2, b2[None, :], W3, b3[None, :])


# speedup vs baseline: 1.1322x; 1.0008x over previous
import functools

import jax
import jax.numpy as jnp
from jax import lax
from jax.experimental import pallas as pl
from jax.experimental.pallas import tpu as pltpu
from jax.experimental.pallas import tpu_sc as plsc

B, L, V, D = 4096, 50, 100000, 300
LANES = 16
NW = 32
ITEMS = B // NW
NG = V * D // LANES
CH = 20
GI = 1000
NSTR = 5
CW = CH * LANES
ACC = 324
NWIN = (D + LANES - 1) // LANES
OFFS = tuple(min(LANES * j, D - LANES) for j in range(NWIN))
INV_L = 1.0 / L
TOFFS = (0, 16, 32, 34)


def _pool_body(x_hbm, tab_hbm, out_hbm, xv, gidx, cbuf, chunks, accbuf, hv,
               sem0, sem1):
    c = lax.axis_index("c")
    s = lax.axis_index("s")
    wid = s * 2 + c
    pltpu.sync_copy(x_hbm.at[pl.ds(wid * (ITEMS * L), ITEMS * L)], xv)
    iota = lax.broadcasted_iota(jnp.int32, (LANES,), 0)
    sems = (sem0, sem1)

    def build(i, k):
        for off in TOFFS:
            t = xv[pl.ds(L * i + off, LANES)]
            m12 = t * 12
            g0 = t * 18 + (m12 >> 4)
            cbuf[k, pl.ds(off, LANES)] = (
                ((m12 & 15) >> 2) * CW).astype(jnp.float32)
            for j in range(CH):
                gidx[k, pl.ds(L * j + off, LANES)] = jnp.minimum(
                    g0 + j, NG - 1)

    def start(i, k):
        pltpu.async_copy(tab_hbm.at[gidx.at[k]], chunks.at[k], sems[k])

    def wait(k):
        pltpu.make_async_copy(tab_hbm.at[gidx.at[k]], chunks.at[k],
                              sems[k]).wait()

    def accumulate(i, k):
        def tbody(r, _):
            bspl = plsc.load_gather(cbuf.at[k], [r + iota * 0])
            bidx = bspl.astype(jnp.int32) + iota
            for j in range(CH):
                v = chunks[k, L * j + r, :]
                plsc.addupdate_scatter(accbuf, [bidx + LANES * j], v)
            return 0

        lax.fori_loop(0, L, tbody, 0, unroll=False)

    def finalize(i):
        for j in range(NWIN):
            acc = accbuf[pl.ds(OFFS[j], LANES)]
            for cc in range(1, 4):
                acc = acc + accbuf[pl.ds(ACC * cc + OFFS[j], LANES)]
            hv[pl.ds(D * i + OFFS[j], LANES)] = acc * INV_L
        z = jnp.zeros((LANES,), jnp.float32)
        for w in range(4 * CW // LANES):
            accbuf[pl.ds(LANES * w, LANES)] = z

    z = jnp.zeros((LANES,), jnp.float32)
    for w in range(4 * CW // LANES):
        accbuf[pl.ds(LANES * w, LANES)] = z

    build(0, 0)
    start(0, 0)

    def gbody(g, _):
        for k in range(2):
            i = 2 * g + k
            wait(k)

            @pl.when(i + 1 < ITEMS)
            def _():
                build(i + 1, 1 - k)
                start(i + 1, 1 - k)

            accumulate(i, k)
            finalize(i)
        return 0

    lax.fori_loop(0, ITEMS // 2, gbody, 0, unroll=False)
    pltpu.sync_copy(hv, out_hbm.at[pl.ds(wid * (ITEMS * D), ITEMS * D)])


def _make_pool():
    mesh = plsc.VectorSubcoreMesh(core_axis_name="c", subcore_axis_name="s")
    return functools.partial(
        pl.kernel,
        mesh=mesh,
        compiler_params=pltpu.CompilerParams(
            use_tc_tiling_on_sc=False, needs_layout_passes=False),
        out_type=jax.ShapeDtypeStruct((B * D,), jnp.float32),
        scratch_types=[
            pltpu.VMEM((ITEMS * L,), jnp.int32),
            pltpu.VMEM((2, GI), jnp.int32),
            pltpu.VMEM((2, 64), jnp.float32),
            pltpu.VMEM((2, GI, LANES), jnp.float32),
            pltpu.VMEM((4 * CW,), jnp.float32),
            pltpu.VMEM((ITEMS * D,), jnp.float32),
            pltpu.SemaphoreType.DMA,
            pltpu.SemaphoreType.DMA,
        ],
    )(_pool_body)


_pool = _make_pool()


def _pooled(x, table):
    x_flat = x.astype(jnp.int32).reshape(B * L)
    tab16 = table.reshape(NG, LANES)
    return _pool(x_flat, tab16).reshape(B, D)


def _mlp_body(h_ref, w1_ref, b1_ref, w2_ref, b2_ref, w3_ref, b3_ref, o_ref):
    h = h_ref[...]
    z = jnp.maximum(jnp.dot(h, w1_ref[...],
                            preferred_element_type=jnp.float32) + b1_ref[...], 0.0)
    z = jnp.maximum(jnp.dot(z, w2_ref[...],
                            preferred_element_type=jnp.float32) + b2_ref[...], 0.0)
    o_ref[...] = jnp.dot(z, w3_ref[...],
                         preferred_element_type=jnp.float32) + b3_ref[...]


def _mlp(h, W1, b1, W2, b2, W3, b3):
    BLK = 512
    grid = (B // BLK,)
    full = lambda shape: pl.BlockSpec(shape, lambda i: (0, 0))
    return pl.pallas_call(
        _mlp_body,
        grid=grid,
        in_specs=[
            pl.BlockSpec((BLK, D), lambda i: (i, 0)),
            full(W1.shape), full(b1.shape), full(W2.shape), full(b2.shape),
            full(W3.shape), full(b3.shape),
        ],
        out_specs=pl.BlockSpec((BLK, 1), lambda i: (i, 0)),
        out_shape=jax.ShapeDtypeStruct((B, 1), jnp.float32),
    )(h, W1, b1, W2, b2, W3, b3)


def kernel(x, table, W1, b1, W2, b2, W3, b3):
    h = _pooled(x, table)
    return _mlp(h, W1, b1[None, :], W2, b2[None, :], W3, b3[None, :])
